# baseline (speedup 1.0000x reference)
import functools

import jax
import jax.numpy as jnp
import numpy as np
from jax import lax
from jax.experimental import pallas as pl
from jax.experimental.pallas import tpu as pltpu
from jax.experimental.pallas import tpu_sc as plsc

_NUM_CORES = 2
_NUM_SUBCORES = 16
_LANES = 16
_NUM_WORKERS = _NUM_CORES * _NUM_SUBCORES

_B, _D, _G = 16384, 11, 10
_ROWS = _B // _NUM_WORKERS
_STEPS = _ROWS // _LANES

_T2_BITS = (0x3C23D70A, 0x3D23D70A, 0x3DB851EC, 0x3E23D70A, 0x3E800000,
            0x3EB851EC, 0x3EFAE146, 0x3F23D70A, 0x3F4F5C2A)
_T2 = tuple(float(np.uint32(b).view(np.float32)) for b in _T2_BITS)


def _vfrho_body(x1_hbm, y1_hbm, x2_hbm, y2_hbm, dgt_hbm, out_hbm,
                x1_v, y1_v, x2_v, y2_v, dg_v, idx_v, out_v, zsem, gsem):
    wid = lax.axis_index("s") * _NUM_CORES + lax.axis_index("c")
    base = wid * _ROWS
    chunk = pl.ds(base, _ROWS)

    z_copies = [
        pltpu.async_copy(x1_hbm.at[chunk], x1_v, zsem),
        pltpu.async_copy(y1_hbm.at[chunk], y1_v, zsem),
        pltpu.async_copy(x2_hbm.at[chunk], x2_v, zsem),
        pltpu.async_copy(y2_hbm.at[chunk], y2_v, zsem),
    ]
    pltpu.make_async_copy(
        dgt_hbm.at[pl.ds(base, _ROWS), :], dg_v, gsem).start()
    for c in z_copies:
        c.wait()

    lane = lax.iota(jnp.int32, _LANES)
    t2 = [jnp.full((_LANES,), v, jnp.float32) for v in _T2]

    def step_bucket(i, carry):
        sl = pl.ds(i * _LANES, _LANES)
        dx = x2_v[sl] - x1_v[sl]
        dy = y2_v[sl] - y1_v[sl]
        r2 = dx * dx + dy * dy
        bucket = jnp.zeros((_LANES,), jnp.int32)
        for c in t2:
            bucket = bucket + (r2 >= c).astype(jnp.int32)
        idx_v[sl] = bucket
        return carry
    lax.fori_loop(0, _STEPS, step_bucket, 0)

    pltpu.make_async_copy(dgt_hbm.at[pl.ds(base, _ROWS), :], dg_v, gsem).wait()

    def step_gather(i, carry):
        sl = pl.ds(i * _LANES, _LANES)
        rows = lane + i * _LANES
        out_v[sl] = plsc.load_gather(dg_v, [rows, idx_v[sl]])
        return carry
    lax.fori_loop(0, _STEPS, step_gather, 0)
    pltpu.sync_copy(out_v, out_hbm.at[chunk])


_vfrho_sc = functools.partial(
    pl.kernel,
    out_type=jax.ShapeDtypeStruct((_B,), jnp.float32),
    mesh=plsc.VectorSubcoreMesh(core_axis_name="c", subcore_axis_name="s"),
    compiler_params=pltpu.CompilerParams(needs_layout_passes=False),
    scratch_types=[
        pltpu.VMEM((_ROWS,), jnp.float32),
        pltpu.VMEM((_ROWS,), jnp.float32),
        pltpu.VMEM((_ROWS,), jnp.float32),
        pltpu.VMEM((_ROWS,), jnp.float32),
        pltpu.VMEM((_ROWS, _G), jnp.float32),
        pltpu.VMEM((_ROWS,), jnp.int32),
        pltpu.VMEM((_ROWS,), jnp.float32),
        pltpu.SemaphoreType.DMA,
        pltpu.SemaphoreType.DMA,
    ],
)(_vfrho_body)


def kernel(z_1, z_2, dist_grade):
    return _vfrho_sc(z_1[:, 0], z_1[:, 2], z_2[:, 0], z_2[:, 2], dist_grade)


# speedup vs baseline: 1.2425x; 1.2425x over previous
import functools

import jax
import jax.numpy as jnp
import numpy as np
from jax import lax
from jax.experimental import pallas as pl
from jax.experimental.pallas import tpu as pltpu
from jax.experimental.pallas import tpu_sc as plsc

_NUM_CORES = 2
_NUM_SUBCORES = 16
_LANES = 16
_NUM_WORKERS = _NUM_CORES * _NUM_SUBCORES

_B, _D, _G = 16384, 11, 10
_ROWS = _B // _NUM_WORKERS
_STEPS = _ROWS // _LANES

_T2_BITS = (0x3C23D70A, 0x3D23D70A, 0x3DB851EC, 0x3E23D70A, 0x3E800000,
            0x3EB851EC, 0x3EFAE146, 0x3F23D70A, 0x3F4F5C2A)
_T2 = tuple(float(np.uint32(b).view(np.float32)) for b in _T2_BITS)


def _vfrho_body(x1_hbm, y1_hbm, x2_hbm, y2_hbm, dgt_hbm, out_hbm,
                x1_v, y1_v, x2_v, y2_v, dg_v, idx_v, out_v, zsem, gsem):
    wid = lax.axis_index("s") * _NUM_CORES + lax.axis_index("c")
    base = wid * _ROWS
    chunk = pl.ds(base, _ROWS)

    z_copies = [
        pltpu.async_copy(x1_hbm.at[chunk], x1_v, zsem),
        pltpu.async_copy(y1_hbm.at[chunk], y1_v, zsem),
        pltpu.async_copy(x2_hbm.at[chunk], x2_v, zsem),
        pltpu.async_copy(y2_hbm.at[chunk], y2_v, zsem),
    ]
    def issue_dg(g, carry):
        pltpu.make_async_copy(
            dgt_hbm.at[pl.ds(g * _B + base, _ROWS)],
            dg_v.at[pl.ds(g * _ROWS, _ROWS)], gsem).start()
        return carry
    lax.fori_loop(0, _G, issue_dg, 0)
    for c in z_copies:
        c.wait()

    lane = lax.iota(jnp.int32, _LANES)
    t2 = [jnp.full((_LANES,), v, jnp.float32) for v in _T2]

    def step_bucket(i, carry):
        sl = pl.ds(i * _LANES, _LANES)
        dx = x2_v[sl] - x1_v[sl]
        dy = y2_v[sl] - y1_v[sl]
        r2 = dx * dx + dy * dy
        bucket = jnp.zeros((_LANES,), jnp.int32)
        for c in t2:
            bucket = bucket + (r2 >= c).astype(jnp.int32)
        idx_v[sl] = bucket * _ROWS + (lane + i * _LANES)
        return carry
    lax.fori_loop(0, _STEPS, step_bucket, 0)

    pltpu.make_async_copy(dgt_hbm.at[pl.ds(0, _G * _ROWS)], dg_v, gsem).wait()

    def step_gather(i, carry):
        sl = pl.ds(i * _LANES, _LANES)
        out_v[sl] = plsc.load_gather(dg_v, [idx_v[sl]])
        return carry
    lax.fori_loop(0, _STEPS, step_gather, 0)
    pltpu.sync_copy(out_v, out_hbm.at[chunk])


_vfrho_sc = functools.partial(
    pl.kernel,
    out_type=jax.ShapeDtypeStruct((_B,), jnp.float32),
    mesh=plsc.VectorSubcoreMesh(core_axis_name="c", subcore_axis_name="s"),
    compiler_params=pltpu.CompilerParams(needs_layout_passes=False),
    scratch_types=[
        pltpu.VMEM((_ROWS,), jnp.float32),
        pltpu.VMEM((_ROWS,), jnp.float32),
        pltpu.VMEM((_ROWS,), jnp.float32),
        pltpu.VMEM((_ROWS,), jnp.float32),
        pltpu.VMEM((_G * _ROWS,), jnp.float32),
        pltpu.VMEM((_ROWS,), jnp.int32),
        pltpu.VMEM((_ROWS,), jnp.float32),
        pltpu.SemaphoreType.DMA,
        pltpu.SemaphoreType.DMA,
    ],
)(_vfrho_body)


def kernel(z_1, z_2, dist_grade):
    dgt = dist_grade.T.reshape(-1)
    return _vfrho_sc(z_1[:, 0], z_1[:, 2], z_2[:, 0], z_2[:, 2], dgt)


# speedup vs baseline: 1.2500x; 1.0061x over previous
import functools

import jax
import jax.numpy as jnp
import numpy as np
from jax import lax
from jax.experimental import pallas as pl
from jax.experimental.pallas import tpu as pltpu
from jax.experimental.pallas import tpu_sc as plsc

_NUM_CORES = 2
_NUM_SUBCORES = 16
_LANES = 16
_NUM_WORKERS = _NUM_CORES * _NUM_SUBCORES

_B, _D, _G = 16384, 11, 10
_ROWS = _B // _NUM_WORKERS
_STEPS = _ROWS // _LANES

_T2_BITS = (0x3C23D70A, 0x3D23D70A, 0x3DB851EC, 0x3E23D70A, 0x3E800000,
            0x3EB851EC, 0x3EFAE146, 0x3F23D70A, 0x3F4F5C2A)
_T2 = tuple(float(np.uint32(b).view(np.float32)) for b in _T2_BITS)


def _vfrho_body(x1_hbm, y1_hbm, x2_hbm, y2_hbm, dgt_hbm, out_hbm,
                x1_v, y1_v, x2_v, y2_v, dg_v, idx_v, out_v, zsem, gsem):
    wid = lax.axis_index("s") * _NUM_CORES + lax.axis_index("c")
    base = wid * _ROWS
    chunk = pl.ds(base, _ROWS)

    z_copies = [
        pltpu.async_copy(x1_hbm.at[chunk], x1_v, zsem),
        pltpu.async_copy(y1_hbm.at[chunk], y1_v, zsem),
        pltpu.async_copy(x2_hbm.at[chunk], x2_v, zsem),
        pltpu.async_copy(y2_hbm.at[chunk], y2_v, zsem),
    ]
    def issue_dg(g, carry):
        pltpu.make_async_copy(
            dgt_hbm.at[pl.ds(g * _B + base, _ROWS)],
            dg_v.at[pl.ds(g * _ROWS, _ROWS)], gsem).start()
        return carry
    lax.fori_loop(0, _G, issue_dg, 0)
    for c in z_copies:
        c.wait()

    lane = lax.iota(jnp.int32, _LANES)
    t2 = [jnp.full((_LANES,), v, jnp.float32) for v in _T2]

    def step_bucket(i, carry):
        sl = pl.ds(i * _LANES, _LANES)
        dx = x2_v[sl] - x1_v[sl]
        dy = y2_v[sl] - y1_v[sl]
        r2 = dx * dx + dy * dy
        bucket = jnp.zeros((_LANES,), jnp.int32)
        for c in t2:
            bucket = bucket + (r2 >= c).astype(jnp.int32)
        idx_v[sl] = bucket * _ROWS + (lane + i * _LANES)
        return carry
    lax.fori_loop(0, _STEPS, step_bucket, 0)

    pltpu.make_async_copy(dgt_hbm.at[pl.ds(0, _G * _ROWS)], dg_v, gsem).wait()

    def step_gather(i, carry):
        sl = pl.ds(i * _LANES, _LANES)
        out_v[sl] = plsc.load_gather(dg_v, [idx_v[sl]])
        return carry
    lax.fori_loop(0, _STEPS, step_gather, 0)
    pltpu.sync_copy(out_v, out_hbm.at[chunk])


_vfrho_sc = functools.partial(
    pl.kernel,
    out_type=jax.ShapeDtypeStruct((_B,), jnp.float32),
    mesh=plsc.VectorSubcoreMesh(core_axis_name="c", subcore_axis_name="s"),
    compiler_params=pltpu.CompilerParams(
        needs_layout_passes=False, allow_input_fusion=[0, 1, 2, 3, 4]),
    scratch_types=[
        pltpu.VMEM((_ROWS,), jnp.float32),
        pltpu.VMEM((_ROWS,), jnp.float32),
        pltpu.VMEM((_ROWS,), jnp.float32),
        pltpu.VMEM((_ROWS,), jnp.float32),
        pltpu.VMEM((_G * _ROWS,), jnp.float32),
        pltpu.VMEM((_ROWS,), jnp.int32),
        pltpu.VMEM((_ROWS,), jnp.float32),
        pltpu.SemaphoreType.DMA,
        pltpu.SemaphoreType.DMA,
    ],
)(_vfrho_body)


def kernel(z_1, z_2, dist_grade):
    dgt = dist_grade.T.reshape(-1)
    return _vfrho_sc(z_1[:, 0], z_1[:, 2], z_2[:, 0], z_2[:, 2], dgt)
